# initial kernel scaffold (unmeasured)
import jax
import jax.numpy as jnp
from jax import lax
from jax.experimental import pallas as pl
from jax.experimental.pallas import tpu as pltpu


N_CHUNKS = 8


def kernel(dy, W):
    M, F = dy.shape
    D = W.shape[0]
    MH = M // 2
    DC = D // N_CHUNKS

    my_y0 = lax.axis_index("y")
    dy_half = lax.dynamic_slice_in_dim(dy, my_y0 * MH, MH, axis=0)

    def body(dy_ref, w_ref, out_ref,
             sx_ref, rx_ref, sy_ref, ry_ref,
             x_send, x_recv, y_send, y_recv):
        c = pl.program_id(0)
        my_x = lax.axis_index("x")
        my_y = lax.axis_index("y")

        @pl.when(c == 0)
        def _barrier():
            bsem = pltpu.get_barrier_semaphore()
            pl.semaphore_signal(
                bsem, inc=1, device_id=(1 - my_x, my_y),
                device_id_type=pl.DeviceIdType.MESH)
            pl.semaphore_signal(
                bsem, inc=1, device_id=(my_x, 1 - my_y),
                device_id_type=pl.DeviceIdType.MESH)
            pl.semaphore_wait(bsem, 2)

        slot = lax.rem(c, 2)

        p = lax.dot_general(
            dy_ref[...], w_ref[...],
            (((1,), (1,)), ((), ())),
            preferred_element_type=jnp.float32)
        sx_ref[slot] = p

        rdma_x = pltpu.make_async_remote_copy(
            src_ref=sx_ref.at[slot], dst_ref=rx_ref.at[slot],
            send_sem=x_send.at[slot], recv_sem=x_recv.at[slot],
            device_id=(1 - my_x, my_y),
            device_id_type=pl.DeviceIdType.MESH)
        rdma_x.start()
        rdma_x.wait()

        r = p + rx_ref[slot]
        sy_ref[slot] = r

        rdma_y = pltpu.make_async_remote_copy(
            src_ref=sy_ref.at[slot], dst_ref=ry_ref.at[slot],
            send_sem=y_send.at[slot], recv_sem=y_recv.at[slot],
            device_id=(my_x, 1 - my_y),
            device_id_type=pl.DeviceIdType.MESH)
        rdma_y.start()
        rdma_y.wait()

        out_ref[pl.ds(my_y * MH, MH), :] = r
        out_ref[pl.ds((1 - my_y) * MH, MH), :] = ry_ref[slot]

    return pl.pallas_call(
        body,
        grid=(N_CHUNKS,),
        out_shape=jax.ShapeDtypeStruct((M, D), jnp.float32),
        in_specs=[
            pl.BlockSpec(memory_space=pltpu.VMEM),
            pl.BlockSpec((DC, F), lambda c: (c, 0)),
        ],
        out_specs=pl.BlockSpec((M, DC), lambda c: (0, c)),
        scratch_shapes=[
            pltpu.VMEM((2, MH, DC), jnp.float32),
            pltpu.VMEM((2, MH, DC), jnp.float32),
            pltpu.VMEM((2, MH, DC), jnp.float32),
            pltpu.VMEM((2, MH, DC), jnp.float32),
            pltpu.SemaphoreType.DMA((2,)),
            pltpu.SemaphoreType.DMA((2,)),
            pltpu.SemaphoreType.DMA((2,)),
            pltpu.SemaphoreType.DMA((2,)),
        ],
        compiler_params=pltpu.CompilerParams(
            collective_id=0,
            dimension_semantics=("arbitrary",),
        ),
    )(dy_half, W)


# baseline (device time: 303855 ns/iter reference)
import jax
import jax.numpy as jnp
from jax import lax
from jax.experimental import pallas as pl
from jax.experimental.pallas import tpu as pltpu


N_CHUNKS = 8


def kernel(dy, W):
    M, F = dy.shape
    D = W.shape[0]
    MH = M // 2
    DC = D // N_CHUNKS

    my_y0 = lax.axis_index("y")
    dy_half = lax.dynamic_slice_in_dim(dy, my_y0 * MH, MH, axis=0)

    def body(dy_ref, w_ref, out_ref,
             sx_ref, rx_ref, sy_ref, ry_ref,
             x_send, x_recv, y_send, y_recv):
        c = pl.program_id(0)
        my_x = lax.axis_index("x")
        my_y = lax.axis_index("y")

        @pl.when(c == 0)
        def _barrier():
            bsem = pltpu.get_barrier_semaphore()
            pl.semaphore_signal(
                bsem, inc=1, device_id=(1 - my_x, my_y),
                device_id_type=pl.DeviceIdType.MESH)
            pl.semaphore_signal(
                bsem, inc=1, device_id=(my_x, 1 - my_y),
                device_id_type=pl.DeviceIdType.MESH)
            pl.semaphore_wait(bsem, 2)

        slot = lax.rem(c, 2)

        p = lax.dot_general(
            dy_ref[...], w_ref[...],
            (((1,), (1,)), ((), ())),
            preferred_element_type=jnp.float32)
        sx_ref[slot] = p

        rdma_x = pltpu.make_async_remote_copy(
            src_ref=sx_ref.at[slot], dst_ref=rx_ref.at[slot],
            send_sem=x_send.at[slot], recv_sem=x_recv.at[slot],
            device_id=(1 - my_x, my_y),
            device_id_type=pl.DeviceIdType.MESH)
        rdma_x.start()
        rdma_x.wait()

        r = p + rx_ref[slot]
        sy_ref[slot] = r

        rdma_y = pltpu.make_async_remote_copy(
            src_ref=sy_ref.at[slot], dst_ref=ry_ref.at[slot],
            send_sem=y_send.at[slot], recv_sem=y_recv.at[slot],
            device_id=(my_x, 1 - my_y),
            device_id_type=pl.DeviceIdType.MESH)
        rdma_y.start()
        rdma_y.wait()

        out_ref[pl.ds(my_y * MH, MH), :] = r
        out_ref[pl.ds((1 - my_y) * MH, MH), :] = ry_ref[slot]

    return pl.pallas_call(
        body,
        grid=(N_CHUNKS,),
        out_shape=jax.ShapeDtypeStruct((M, D), jnp.float32),
        in_specs=[
            pl.BlockSpec(memory_space=pltpu.VMEM),
            pl.BlockSpec((DC, F), lambda c: (c, 0)),
        ],
        out_specs=pl.BlockSpec((M, DC), lambda c: (0, c)),
        scratch_shapes=[
            pltpu.VMEM((2, MH, DC), jnp.float32),
            pltpu.VMEM((2, MH, DC), jnp.float32),
            pltpu.VMEM((2, MH, DC), jnp.float32),
            pltpu.VMEM((2, MH, DC), jnp.float32),
            pltpu.SemaphoreType.DMA((2,)),
            pltpu.SemaphoreType.DMA((2,)),
            pltpu.SemaphoreType.DMA((2,)),
            pltpu.SemaphoreType.DMA((2,)),
        ],
        compiler_params=pltpu.CompilerParams(
            collective_id=0,
            dimension_semantics=("arbitrary",),
            vmem_limit_bytes=64 * 1024 * 1024,
        ),
    )(dy_half, W)


# device time: 164252 ns/iter; 1.8499x vs baseline; 1.8499x over previous
import jax
import jax.numpy as jnp
from jax import lax
from jax.experimental import pallas as pl
from jax.experimental.pallas import tpu as pltpu


N_CHUNKS = 8
GRID = N_CHUNKS + 2


def kernel(dy, W):
    M, F = dy.shape
    D = W.shape[0]
    MH = M // 2
    DC = D // N_CHUNKS

    my_y0 = lax.axis_index("y")
    dy_half = lax.dynamic_slice_in_dim(dy, my_y0 * MH, MH, axis=0)

    def body(dy_ref, w_ref, out_ref,
             sx_ref, rx_ref, sy_ref, ry_ref,
             x_send, x_recv, y_send, y_recv,
             credit_x, credit_y):
        c = pl.program_id(0)
        my_x = lax.axis_index("x")
        my_y = lax.axis_index("y")
        x_nbr = (1 - my_x, my_y)
        y_nbr = (my_x, 1 - my_y)

        def x_rdma(slot):
            return pltpu.make_async_remote_copy(
                src_ref=sx_ref.at[slot], dst_ref=rx_ref.at[slot],
                send_sem=x_send.at[slot], recv_sem=x_recv.at[slot],
                device_id=x_nbr, device_id_type=pl.DeviceIdType.MESH)

        def y_rdma(slot):
            return pltpu.make_async_remote_copy(
                src_ref=sy_ref.at[slot], dst_ref=ry_ref.at[slot],
                send_sem=y_send.at[slot], recv_sem=y_recv.at[slot],
                device_id=y_nbr, device_id_type=pl.DeviceIdType.MESH)

        @pl.when(c == 0)
        def _barrier():
            bsem = pltpu.get_barrier_semaphore()
            pl.semaphore_signal(
                bsem, inc=1, device_id=x_nbr,
                device_id_type=pl.DeviceIdType.MESH)
            pl.semaphore_signal(
                bsem, inc=1, device_id=y_nbr,
                device_id_type=pl.DeviceIdType.MESH)
            pl.semaphore_wait(bsem, 2)

        @pl.when(c < N_CHUNKS)
        def _phase_a():
            slot = lax.rem(c, 2)
            p = lax.dot_general(
                dy_ref[...], w_ref[...],
                (((1,), (1,)), ((), ())),
                preferred_element_type=jnp.float32)

            @pl.when(c >= 2)
            def _():
                x_rdma(slot).wait_send()
                pl.semaphore_wait(credit_x, 1)

            sx_ref[slot] = p
            x_rdma(slot).start()

        @pl.when((c >= 1) & (c <= N_CHUNKS))
        def _phase_b():
            slot = lax.rem(c - 1, 2)
            x_rdma(slot).wait_recv()
            r = sx_ref[slot] + rx_ref[slot]

            @pl.when(c >= 3)
            def _():
                y_rdma(slot).wait_send()
                pl.semaphore_wait(credit_y, 1)

            sy_ref[slot] = r
            y_rdma(slot).start()

            @pl.when(c <= 6)
            def _():
                pl.semaphore_signal(
                    credit_x, inc=1, device_id=x_nbr,
                    device_id_type=pl.DeviceIdType.MESH)

        @pl.when(c >= 2)
        def _phase_c():
            slot = lax.rem(c - 2, 2)
            y_rdma(slot).wait_recv()
            out_ref[pl.ds(my_y * MH, MH), :] = sy_ref[slot]
            out_ref[pl.ds((1 - my_y) * MH, MH), :] = ry_ref[slot]

            @pl.when(c <= 7)
            def _():
                pl.semaphore_signal(
                    credit_y, inc=1, device_id=y_nbr,
                    device_id_type=pl.DeviceIdType.MESH)

        @pl.when(c == GRID - 1)
        def _drain():
            for s in (0, 1):
                x_rdma(s).wait_send()
                y_rdma(s).wait_send()

    return pl.pallas_call(
        body,
        grid=(GRID,),
        out_shape=jax.ShapeDtypeStruct((M, D), jnp.float32),
        in_specs=[
            pl.BlockSpec(memory_space=pltpu.VMEM),
            pl.BlockSpec(
                (DC, F), lambda c: (jnp.minimum(c, N_CHUNKS - 1), 0)),
        ],
        out_specs=pl.BlockSpec(
            (M, DC), lambda c: (0, jnp.clip(c - 2, 0, N_CHUNKS - 1))),
        scratch_shapes=[
            pltpu.VMEM((2, MH, DC), jnp.float32),
            pltpu.VMEM((2, MH, DC), jnp.float32),
            pltpu.VMEM((2, MH, DC), jnp.float32),
            pltpu.VMEM((2, MH, DC), jnp.float32),
            pltpu.SemaphoreType.DMA((2,)),
            pltpu.SemaphoreType.DMA((2,)),
            pltpu.SemaphoreType.DMA((2,)),
            pltpu.SemaphoreType.DMA((2,)),
            pltpu.SemaphoreType.REGULAR,
            pltpu.SemaphoreType.REGULAR,
        ],
        compiler_params=pltpu.CompilerParams(
            collective_id=0,
            dimension_semantics=("arbitrary",),
            vmem_limit_bytes=64 * 1024 * 1024,
        ),
    )(dy_half, W)
